# baseline (speedup 1.0000x reference)
import functools
import math

import jax
import jax.numpy as jnp
from jax.experimental import pallas as pl
from jax.experimental.pallas import tpu as pltpu

NEG = float(jnp.finfo(jnp.float32).min)
_VMEM_LIMIT = 64 * 1024 * 1024


def _sin_pe(steps, d_model):
    pos = jnp.arange(steps, dtype=jnp.float32)[:, None]
    div = jnp.exp(jnp.arange(0, d_model, 2, dtype=jnp.float32)
                  * (-math.log(10000.0) / d_model))
    pe = jnp.zeros((steps, d_model), jnp.float32)
    pe = pe.at[:, 0::2].set(jnp.sin(pos * div))
    pe = pe.at[:, 1::2].set(jnp.cos(pos * div))
    return pe


def _encode_kernel(u_ref, x_ref, m_ref, nembt_ref, pet_ref,
                   ulw_ref, ulb_ref, w1_ref, b1_ref, w2_ref, b2_ref,
                   w3_ref, b3_ref, hw1_ref, hb1_ref, hw2_ref, hb2_ref,
                   g_ref, beta_ref, o_ref, *, s_steps, n_nodes, eps=1e-5):
    S, N = s_steps, n_nodes
    H = g_ref.shape[-1]
    T = S * N

    u = u_ref[0]
    xl = jnp.dot(u, ulw_ref[...], preferred_element_type=jnp.float32) + ulb_ref[...]
    xlt = jnp.broadcast_to(xl[:, None, :], (S, N, H)).reshape(T, H)
    z = xlt + nembt_ref[...]
    z = jnp.where(z > 0.0, z, 0.01 * z)
    z = jnp.maximum(jnp.dot(z, w1_ref[...], preferred_element_type=jnp.float32)
                    + b1_ref[...], 0.0)
    z = jnp.maximum(jnp.dot(z, w2_ref[...], preferred_element_type=jnp.float32)
                    + b2_ref[...], 0.0)
    z = jnp.dot(z, w3_ref[...], preferred_element_type=jnp.float32) + b3_ref[...]
    q = z + pet_ref[...]

    m = m_ref[0]
    xm = x_ref[0] * m
    h1 = jnp.maximum(xm * hw1_ref[...] + hb1_ref[...], 0.0)
    h2 = jnp.maximum(jnp.dot(h1, hw2_ref[...], preferred_element_type=jnp.float32)
                     + hb2_ref[...], 0.0)

    h = jnp.where(m > 0.0, h2 + q, q)
    mu = jnp.mean(h, axis=-1, keepdims=True)
    var = jnp.mean((h - mu) ** 2, axis=-1, keepdims=True)
    o_ref[0] = (h - mu) * jax.lax.rsqrt(var + eps) * g_ref[...] + beta_ref[...]


def _encode(x, mask, u, p, S, N):
    B = x.shape[0]
    T = S * N
    H = p["h_norm_g"].shape[-1]
    Cu = u.shape[-1]
    Cp = max(8, ((Cu + 7) // 8) * 8)
    u_pad = jnp.zeros((B, S, Cp), jnp.float32).at[:, :, :Cu].set(
        u.astype(jnp.float32))
    ulw = jnp.zeros((Cp, H), jnp.float32).at[:Cu].set(p["u_lin_w"])
    x3 = x.reshape(B, T, 1).astype(jnp.float32)
    m3 = mask.reshape(B, T, 1).astype(jnp.float32)
    nembt = jnp.tile(p["u_node_emb"], (S, 1))
    pet = jnp.repeat(_sin_pe(S, H), N, axis=0)

    row = lambda b: (b, 0, 0)
    c2 = lambda b: (0, 0)

    return pl.pallas_call(
        functools.partial(_encode_kernel, s_steps=S, n_nodes=N),
        out_shape=jax.ShapeDtypeStruct((B, T, H), jnp.float32),
        grid=(B,),
        in_specs=[
            pl.BlockSpec((1, S, Cp), row),
            pl.BlockSpec((1, T, 1), row),
            pl.BlockSpec((1, T, 1), row),
            pl.BlockSpec((T, H), c2), pl.BlockSpec((T, H), c2),
            pl.BlockSpec((Cp, H), c2), pl.BlockSpec((1, H), c2),
            pl.BlockSpec((H, H), c2), pl.BlockSpec((1, H), c2),
            pl.BlockSpec((H, H), c2), pl.BlockSpec((1, H), c2),
            pl.BlockSpec((H, H), c2), pl.BlockSpec((1, H), c2),
            pl.BlockSpec((1, H), c2), pl.BlockSpec((1, H), c2),
            pl.BlockSpec((H, H), c2), pl.BlockSpec((1, H), c2),
            pl.BlockSpec((1, H), c2), pl.BlockSpec((1, H), c2),
        ],
        out_specs=pl.BlockSpec((1, T, H), row),
        compiler_params=pltpu.CompilerParams(
            dimension_semantics=("parallel",), vmem_limit_bytes=_VMEM_LIMIT),
    )(u_pad, x3, m3, nembt, pet,
      ulw, p["u_lin_b"],
      p["u_mlp_w1"], p["u_mlp_b1"], p["u_mlp_w2"], p["u_mlp_b2"],
      p["u_mlp_w3"], p["u_mlp_b3"],
      p["h_enc_w1"], p["h_enc_b1"], p["h_enc_w2"], p["h_enc_b2"],
      p["h_norm_g"], p["h_norm_b"])


def _layer_kernel(hq_ref, hk_ref, xq_ref, xk_ref, mq_ref, mk_ref,
                  ohq_ref, ohk_ref, adj_ref,
                  wx_ref, bx_ref, wqs_ref, bqs_ref, wk_ref, bk_ref,
                  w1b_ref, b1_ref, w2_ref, b2_ref, apad_ref,
                  lng_ref, lnb_ref,
                  rw1_ref, rb1_ref, rw2_ref, rb2_ref, rw3_ref, rb3_ref,
                  vembt_ref, membt_ref,
                  ho_ref, imp_ref,
                  mt_sc, lt_sc, st_sc, ms_sc, ls_sc, ss_sc,
                  *, hidden, mask_spatial, add_emb, eps=1e-5):
    H = hidden
    ki = pl.program_id(2)

    @pl.when(ki == 0)
    def _init():
        mt_sc[...] = jnp.full_like(mt_sc, NEG)
        ms_sc[...] = jnp.full_like(ms_sc, NEG)
        lt_sc[...] = jnp.zeros_like(lt_sc)
        ls_sc[...] = jnp.zeros_like(ls_sc)
        st_sc[...] = jnp.zeros_like(st_sc)
        ss_sc[...] = jnp.zeros_like(ss_sc)

    xq, mq = xq_ref[0], mq_ref[0]
    xk, mk = xk_ref[0], mk_ref[0]
    hq = hq_ref[0] + (xq * wx_ref[...] + bx_ref[...]) * mq
    hk = hk_ref[0] + (xk * wx_ref[...] + bx_ref[...]) * mk
    if add_emb:
        hq = hq + jnp.where(mq > 0.0, vembt_ref[...], membt_ref[...])
        hk = hk + jnp.where(mk > 0.0, vembt_ref[...], membt_ref[...])

    pq = jnp.dot(hq, wqs_ref[...], preferred_element_type=jnp.float32) + bqs_ref[...]
    qv, skipv = pq[:, :H], pq[:, H:]
    kv = jnp.dot(hk, wk_ref[...], preferred_element_type=jnp.float32) + bk_ref[...]

    TQ, TK = qv.shape[0], kv.shape[0]
    e = qv[:, None, :] + kv[None, :, :]
    e = jnp.where(e > 0.0, e, 0.01 * e)
    scores = jnp.sum(e * apad_ref[...][None], axis=-1)
    ebf = e.astype(jnp.bfloat16).reshape(TQ * TK, H)
    m1bf = jnp.maximum(
        jnp.dot(ebf, w1b_ref[...], preferred_element_type=jnp.float32)
        + b1_ref[...], 0.0).astype(jnp.bfloat16).reshape(TQ, TK, H)

    mkb = mk.astype(jnp.bfloat16)
    ohq = ohq_ref[...]
    ohk = ohk_ref[...]
    ohk_obs = ohk * mkb
    mask_t = jnp.einsum("qp,kp->qk", ohq, ohk_obs,
                        preferred_element_type=jnp.float32)
    ohq_adj = jnp.dot(ohq, adj_ref[...],
                      preferred_element_type=jnp.float32).astype(jnp.bfloat16)
    ohk_sp = ohk_obs if mask_spatial else ohk
    mask_s = jnp.einsum("qp,kp->qk", ohq_adj, ohk_sp,
                        preferred_element_type=jnp.float32)

    def update(msk, m_sc, l_sc, s_sc):
        s = jnp.where(msk > 0.0, scores, NEG)
        m_new = jnp.maximum(m_sc[...], jnp.max(s, axis=-1, keepdims=True))
        alpha = jnp.exp(m_sc[...] - m_new)
        p = jnp.where(msk > 0.0, jnp.exp(s - m_new), 0.0)
        l_sc[...] = alpha * l_sc[...] + jnp.sum(p, axis=-1, keepdims=True)
        p_hi = p.astype(jnp.bfloat16)
        p_lo = (p - p_hi.astype(jnp.float32)).astype(jnp.bfloat16)
        dn = (((1,), (1,)), ((0,), (0,)))
        acc = (jax.lax.dot_general(p_hi, m1bf, dn,
                                   preferred_element_type=jnp.float32)
               + jax.lax.dot_general(p_lo, m1bf, dn,
                                     preferred_element_type=jnp.float32))
        s_sc[...] = alpha * s_sc[...] + acc
        m_sc[...] = m_new

    update(mask_t, mt_sc, lt_sc, st_sc)
    update(mask_s, ms_sc, ls_sc, ss_sc)

    @pl.when(ki == pl.num_programs(2) - 1)
    def _done():
        nt = jnp.maximum(lt_sc[...], 1e-9)
        ns = jnp.maximum(ls_sc[...], 1e-9)
        at = (jnp.dot(st_sc[...], w2_ref[...], preferred_element_type=jnp.float32)
              + b2_ref[...] * lt_sc[...]) / nt
        as_ = (jnp.dot(ss_sc[...], w2_ref[...], preferred_element_type=jnp.float32)
               + b2_ref[...] * ls_sc[...]) / ns
        out = at + as_ + skipv
        mu = jnp.mean(out, axis=-1, keepdims=True)
        var = jnp.mean((out - mu) ** 2, axis=-1, keepdims=True)
        h_new = (out - mu) * jax.lax.rsqrt(var + eps) * lng_ref[...] + lnb_ref[...]
        ho_ref[0] = h_new

        r = jnp.maximum(jnp.dot(h_new, rw1_ref[...], preferred_element_type=jnp.float32)
                        + rb1_ref[...], 0.0)
        r = jnp.maximum(jnp.dot(r, rw2_ref[...], preferred_element_type=jnp.float32)
                        + rb2_ref[...], 0.0)
        imp_ref[0] = jnp.sum(r * rw3_ref[...], axis=-1, keepdims=True) + rb3_ref[...]


def _spin_layer(h, x_col, m_col, onehot, adj_pad, vembt, membt, lp,
                *, mask_spatial, add_emb, tq, tk):
    B, T, H = h.shape
    NP = adj_pad.shape[0]
    Cout = lp["ro_b3"].shape[-1]
    nq, nk = T // tq, T // tk

    wqs = jnp.concatenate([lp["wq"], lp["wskip"]], axis=1)
    bqs = jnp.concatenate([lp["bq"], lp["bskip"]], axis=1)
    w1b = lp["w1"].astype(jnp.bfloat16)
    apad = lp["a"]

    q3 = lambda b, qi, ki: (b, qi, 0)
    k3 = lambda b, qi, ki: (b, ki, 0)
    q2 = lambda b, qi, ki: (qi, 0)
    k2 = lambda b, qi, ki: (ki, 0)
    c2 = lambda b, qi, ki: (0, 0)

    body = functools.partial(_layer_kernel, hidden=H,
                             mask_spatial=mask_spatial, add_emb=add_emb)

    return pl.pallas_call(
        body,
        out_shape=(jax.ShapeDtypeStruct((B, T, H), jnp.float32),
                   jax.ShapeDtypeStruct((B, T, Cout), jnp.float32)),
        grid=(B, nq, nk),
        in_specs=[
            pl.BlockSpec((1, tq, H), q3), pl.BlockSpec((1, tk, H), k3),
            pl.BlockSpec((1, tq, 1), q3), pl.BlockSpec((1, tk, 1), k3),
            pl.BlockSpec((1, tq, 1), q3), pl.BlockSpec((1, tk, 1), k3),
            pl.BlockSpec((tq, NP), q2), pl.BlockSpec((tk, NP), k2),
            pl.BlockSpec((NP, NP), c2),
            pl.BlockSpec((1, H), c2), pl.BlockSpec((1, H), c2),
            pl.BlockSpec((H, 2 * H), c2), pl.BlockSpec((1, 2 * H), c2),
            pl.BlockSpec((H, H), c2), pl.BlockSpec((1, H), c2),
            pl.BlockSpec((H, H), c2), pl.BlockSpec((1, H), c2),
            pl.BlockSpec((H, H), c2), pl.BlockSpec((1, H), c2),
            pl.BlockSpec((1, H), c2),
            pl.BlockSpec((1, H), c2), pl.BlockSpec((1, H), c2),
            pl.BlockSpec((H, H), c2), pl.BlockSpec((1, H), c2),
            pl.BlockSpec((H, H), c2), pl.BlockSpec((1, H), c2),
            pl.BlockSpec((1, H), c2), pl.BlockSpec((1, Cout), c2),
            pl.BlockSpec((tq, H), c2), pl.BlockSpec((tq, H), c2),
        ],
        out_specs=(pl.BlockSpec((1, tq, H), q3),
                   pl.BlockSpec((1, tq, Cout), q3)),
        scratch_shapes=[
            pltpu.VMEM((tq, 1), jnp.float32), pltpu.VMEM((tq, 1), jnp.float32),
            pltpu.VMEM((tq, H), jnp.float32),
            pltpu.VMEM((tq, 1), jnp.float32), pltpu.VMEM((tq, 1), jnp.float32),
            pltpu.VMEM((tq, H), jnp.float32),
        ],
        compiler_params=pltpu.CompilerParams(
            dimension_semantics=("parallel", "parallel", "arbitrary"),
            vmem_limit_bytes=_VMEM_LIMIT),
    )(h, h, x_col, x_col, m_col, m_col, onehot, onehot, adj_pad,
      lp["xskip_w"], lp["xskip_b"], wqs, bqs, lp["wk"], lp["bk"],
      w1b, lp["b1"], lp["w2"], lp["b2"], apad,
      lp["ln_g"], lp["ln_b"],
      lp["ro_w1"], lp["ro_b1"], lp["ro_w2"], lp["ro_b2"],
      lp["ro_w3"], lp["ro_b3"], vembt, membt)


def _forward(p, x, u, mask, adj, *, n_layers=4, eta=3):
    B, S, N, _ = x.shape
    H = p["h_norm_g"].shape[-1]
    T = S * N

    h = _encode(x, mask, u, p, S, N)

    NP = max(128, ((N + 127) // 128) * 128)
    node_id = jnp.arange(T, dtype=jnp.int32) % N
    onehot = jax.nn.one_hot(node_id, NP, dtype=jnp.bfloat16)
    adj_pad = jnp.zeros((NP, NP), jnp.float32).at[:N, :N].set(
        adj.astype(jnp.float32)).astype(jnp.bfloat16)

    x_col = x.reshape(B, T, 1).astype(jnp.float32)
    m_col = mask.astype(jnp.float32).reshape(B, T, 1)

    tq = 128 if T % 128 == 0 else T
    tk = 128 if T % 128 == 0 else T
    vembt = jnp.tile(p["valid_emb"], (tq // N if tq % N == 0 else 1, 1))[:tq]
    membt = jnp.tile(p["mask_emb"], (tq // N if tq % N == 0 else 1, 1))[:tq]

    imputations = []
    for l in range(n_layers):
        h, r = _spin_layer(h, x_col, m_col, onehot, adj_pad, vembt, membt,
                           p["layers"][l], mask_spatial=(l < eta),
                           add_emb=(l == eta), tq=tq, tk=tk)
        imputations.append(r.reshape(B, S, N, -1))

    x_hat = imputations.pop(-1)
    return x_hat, imputations


def kernel(x, u, mask, adj,
           u_lin_w, u_lin_b, u_node_emb,
           u_mlp_w1, u_mlp_b1, u_mlp_w2, u_mlp_b2, u_mlp_w3, u_mlp_b3,
           h_enc_w1, h_enc_b1, h_enc_w2, h_enc_b2,
           h_norm_g, h_norm_b, valid_emb, mask_emb,
           l0_xskip_w, l0_xskip_b, l0_wq, l0_bq, l0_wk, l0_bk, l0_w1, l0_b1, l0_w2, l0_b2,
           l0_a, l0_wskip, l0_bskip, l0_ln_g, l0_ln_b,
           l0_ro_w1, l0_ro_b1, l0_ro_w2, l0_ro_b2, l0_ro_w3, l0_ro_b3,
           l1_xskip_w, l1_xskip_b, l1_wq, l1_bq, l1_wk, l1_bk, l1_w1, l1_b1, l1_w2, l1_b2,
           l1_a, l1_wskip, l1_bskip, l1_ln_g, l1_ln_b,
           l1_ro_w1, l1_ro_b1, l1_ro_w2, l1_ro_b2, l1_ro_w3, l1_ro_b3,
           l2_xskip_w, l2_xskip_b, l2_wq, l2_bq, l2_wk, l2_bk, l2_w1, l2_b1, l2_w2, l2_b2,
           l2_a, l2_wskip, l2_bskip, l2_ln_g, l2_ln_b,
           l2_ro_w1, l2_ro_b1, l2_ro_w2, l2_ro_b2, l2_ro_w3, l2_ro_b3,
           l3_xskip_w, l3_xskip_b, l3_wq, l3_bq, l3_wk, l3_bk, l3_w1, l3_b1, l3_w2, l3_b2,
           l3_a, l3_wskip, l3_bskip, l3_ln_g, l3_ln_b,
           l3_ro_w1, l3_ro_b1, l3_ro_w2, l3_ro_b2, l3_ro_w3, l3_ro_b3):
    def _layer(pp):
        return {"xskip_w": pp[0], "xskip_b": pp[1], "wq": pp[2], "bq": pp[3],
                "wk": pp[4], "bk": pp[5], "w1": pp[6], "b1": pp[7], "w2": pp[8],
                "b2": pp[9], "a": pp[10], "wskip": pp[11], "bskip": pp[12],
                "ln_g": pp[13], "ln_b": pp[14], "ro_w1": pp[15], "ro_b1": pp[16],
                "ro_w2": pp[17], "ro_b2": pp[18], "ro_w3": pp[19], "ro_b3": pp[20]}

    p = {
        "u_lin_w": u_lin_w, "u_lin_b": u_lin_b, "u_node_emb": u_node_emb,
        "u_mlp_w1": u_mlp_w1, "u_mlp_b1": u_mlp_b1, "u_mlp_w2": u_mlp_w2,
        "u_mlp_b2": u_mlp_b2, "u_mlp_w3": u_mlp_w3, "u_mlp_b3": u_mlp_b3,
        "h_enc_w1": h_enc_w1, "h_enc_b1": h_enc_b1, "h_enc_w2": h_enc_w2,
        "h_enc_b2": h_enc_b2, "h_norm_g": h_norm_g, "h_norm_b": h_norm_b,
        "valid_emb": valid_emb, "mask_emb": mask_emb,
        "layers": [
            _layer([l0_xskip_w, l0_xskip_b, l0_wq, l0_bq, l0_wk, l0_bk, l0_w1,
                    l0_b1, l0_w2, l0_b2, l0_a, l0_wskip, l0_bskip, l0_ln_g,
                    l0_ln_b, l0_ro_w1, l0_ro_b1, l0_ro_w2, l0_ro_b2, l0_ro_w3,
                    l0_ro_b3]),
            _layer([l1_xskip_w, l1_xskip_b, l1_wq, l1_bq, l1_wk, l1_bk, l1_w1,
                    l1_b1, l1_w2, l1_b2, l1_a, l1_wskip, l1_bskip, l1_ln_g,
                    l1_ln_b, l1_ro_w1, l1_ro_b1, l1_ro_w2, l1_ro_b2, l1_ro_w3,
                    l1_ro_b3]),
            _layer([l2_xskip_w, l2_xskip_b, l2_wq, l2_bq, l2_wk, l2_bk, l2_w1,
                    l2_b1, l2_w2, l2_b2, l2_a, l2_wskip, l2_bskip, l2_ln_g,
                    l2_ln_b, l2_ro_w1, l2_ro_b1, l2_ro_w2, l2_ro_b2, l2_ro_w3,
                    l2_ro_b3]),
            _layer([l3_xskip_w, l3_xskip_b, l3_wq, l3_bq, l3_wk, l3_bk, l3_w1,
                    l3_b1, l3_w2, l3_b2, l3_a, l3_wskip, l3_bskip, l3_ln_g,
                    l3_ln_b, l3_ro_w1, l3_ro_b1, l3_ro_w2, l3_ro_b2, l3_ro_w3,
                    l3_ro_b3]),
        ],
    }
    return _forward(p, x, u, mask, adj, n_layers=4, eta=3)


# speedup vs baseline: 1.6190x; 1.6190x over previous
import functools
import math

import jax
import jax.numpy as jnp
from jax.experimental import pallas as pl
from jax.experimental.pallas import tpu as pltpu

NEG = float(jnp.finfo(jnp.float32).min)
_VMEM_LIMIT = 64 * 1024 * 1024


def _sin_pe(steps, d_model):
    pos = jnp.arange(steps, dtype=jnp.float32)[:, None]
    div = jnp.exp(jnp.arange(0, d_model, 2, dtype=jnp.float32)
                  * (-math.log(10000.0) / d_model))
    pe = jnp.zeros((steps, d_model), jnp.float32)
    pe = pe.at[:, 0::2].set(jnp.sin(pos * div))
    pe = pe.at[:, 1::2].set(jnp.cos(pos * div))
    return pe


def _encode_kernel(u_ref, x_ref, m_ref, nembt_ref, pet_ref,
                   ulw_ref, ulb_ref, w1_ref, b1_ref, w2_ref, b2_ref,
                   w3_ref, b3_ref, hw1_ref, hb1_ref, hw2_ref, hb2_ref,
                   g_ref, beta_ref, o_ref, *, s_steps, n_nodes, eps=1e-5):
    S, N = s_steps, n_nodes
    H = g_ref.shape[-1]
    T = S * N

    u = u_ref[0]
    xl = jnp.dot(u, ulw_ref[...], preferred_element_type=jnp.float32) + ulb_ref[...]
    xlt = jnp.broadcast_to(xl[:, None, :], (S, N, H)).reshape(T, H)
    z = xlt + nembt_ref[...]
    z = jnp.where(z > 0.0, z, 0.01 * z)
    z = jnp.maximum(jnp.dot(z, w1_ref[...], preferred_element_type=jnp.float32)
                    + b1_ref[...], 0.0)
    z = jnp.maximum(jnp.dot(z, w2_ref[...], preferred_element_type=jnp.float32)
                    + b2_ref[...], 0.0)
    z = jnp.dot(z, w3_ref[...], preferred_element_type=jnp.float32) + b3_ref[...]
    q = z + pet_ref[...]

    m = m_ref[0]
    xm = x_ref[0] * m
    h1 = jnp.maximum(xm * hw1_ref[...] + hb1_ref[...], 0.0)
    h2 = jnp.maximum(jnp.dot(h1, hw2_ref[...], preferred_element_type=jnp.float32)
                     + hb2_ref[...], 0.0)

    h = jnp.where(m > 0.0, h2 + q, q)
    mu = jnp.mean(h, axis=-1, keepdims=True)
    var = jnp.mean((h - mu) ** 2, axis=-1, keepdims=True)
    o_ref[0] = (h - mu) * jax.lax.rsqrt(var + eps) * g_ref[...] + beta_ref[...]


def _encode(x, mask, u, p, S, N):
    B = x.shape[0]
    T = S * N
    H = p["h_norm_g"].shape[-1]
    Cu = u.shape[-1]
    Cp = max(8, ((Cu + 7) // 8) * 8)
    u_pad = jnp.zeros((B, S, Cp), jnp.float32).at[:, :, :Cu].set(
        u.astype(jnp.float32))
    ulw = jnp.zeros((Cp, H), jnp.float32).at[:Cu].set(p["u_lin_w"])
    x3 = x.reshape(B, T, 1).astype(jnp.float32)
    m3 = mask.reshape(B, T, 1).astype(jnp.float32)
    nembt = jnp.tile(p["u_node_emb"], (S, 1))
    pet = jnp.repeat(_sin_pe(S, H), N, axis=0)

    row = lambda b: (b, 0, 0)
    c2 = lambda b: (0, 0)

    return pl.pallas_call(
        functools.partial(_encode_kernel, s_steps=S, n_nodes=N),
        out_shape=jax.ShapeDtypeStruct((B, T, H), jnp.float32),
        grid=(B,),
        in_specs=[
            pl.BlockSpec((1, S, Cp), row),
            pl.BlockSpec((1, T, 1), row),
            pl.BlockSpec((1, T, 1), row),
            pl.BlockSpec((T, H), c2), pl.BlockSpec((T, H), c2),
            pl.BlockSpec((Cp, H), c2), pl.BlockSpec((1, H), c2),
            pl.BlockSpec((H, H), c2), pl.BlockSpec((1, H), c2),
            pl.BlockSpec((H, H), c2), pl.BlockSpec((1, H), c2),
            pl.BlockSpec((H, H), c2), pl.BlockSpec((1, H), c2),
            pl.BlockSpec((1, H), c2), pl.BlockSpec((1, H), c2),
            pl.BlockSpec((H, H), c2), pl.BlockSpec((1, H), c2),
            pl.BlockSpec((1, H), c2), pl.BlockSpec((1, H), c2),
        ],
        out_specs=pl.BlockSpec((1, T, H), row),
        compiler_params=pltpu.CompilerParams(
            dimension_semantics=("parallel",), vmem_limit_bytes=_VMEM_LIMIT),
    )(u_pad, x3, m3, nembt, pet,
      ulw, p["u_lin_b"],
      p["u_mlp_w1"], p["u_mlp_b1"], p["u_mlp_w2"], p["u_mlp_b2"],
      p["u_mlp_w3"], p["u_mlp_b3"],
      p["h_enc_w1"], p["h_enc_b1"], p["h_enc_w2"], p["h_enc_b2"],
      p["h_norm_g"], p["h_norm_b"])


def _layer_kernel(adj_sm, h_ref, x_ref, m_ref,
                  wx_ref, bx_ref, wqs_ref, bqs_ref, wk_ref, bk_ref,
                  w1_ref, b1_ref, w2_ref, b2_ref, a_ref,
                  lng_ref, lnb_ref,
                  rw1_ref, rb1_ref, rw2_ref, rb2_ref, rw3_ref, rb3_ref,
                  vembq_ref, membq_ref, vembk_ref, membk_ref,
                  ho_ref, imp_ref,
                  mt_sc, lt_sc, st_sc, ms_sc, ls_sc, ss_sc,
                  *, hidden, gb, s_steps, n_nodes, mask_spatial, add_emb,
                  eps=1e-5):
    H, GB, S, N = hidden, gb, s_steps, n_nodes
    nq = pl.program_id(1)
    nk = pl.program_id(2)
    adj_v = adj_sm[nq * N + nk]
    diag = nq == nk

    @pl.when(nk == 0)
    def _init():
        mt_sc[...] = jnp.full_like(mt_sc, NEG)
        ms_sc[...] = jnp.full_like(ms_sc, NEG)
        lt_sc[...] = jnp.zeros_like(lt_sc)
        ls_sc[...] = jnp.zeros_like(ls_sc)
        st_sc[...] = jnp.zeros_like(st_sc)
        ss_sc[...] = jnp.zeros_like(ss_sc)

    def build_h(n_idx, vm_gate):
        xb = x_ref[:, pl.ds(n_idx * S, S), :].reshape(GB * S, 1)
        mb = m_ref[:, pl.ds(n_idx * S, S), :].reshape(GB * S, 1)
        hb = h_ref[:, pl.ds(n_idx * S, S), :].reshape(GB * S, H)
        hb = hb + (xb * wx_ref[...] + bx_ref[...]) * mb
        if add_emb:
            hb = hb + jnp.where(mb > 0.0, vm_gate[0], vm_gate[1])
        return hb, mb

    vmq = (vembq_ref[0], membq_ref[0]) if add_emb else (None, None)
    vmk = (vembk_ref[0], membk_ref[0]) if add_emb else (None, None)

    @pl.when((adj_v > 0) | diag)
    def _compute():
        hq, _ = build_h(nq, vmq)
        hk, mk = build_h(nk, vmk)
        qv = (jnp.dot(hq, wqs_ref[:, :H],
                      preferred_element_type=jnp.float32) + bqs_ref[:, :H])
        kv = jnp.dot(hk, wk_ref[...], preferred_element_type=jnp.float32) \
            + bk_ref[...]

        e = (qv.reshape(GB, S, H)[:, :, None, :]
             + kv.reshape(GB, S, H)[:, None, :, :])
        e = jnp.where(e > 0.0, e, 0.01 * e)
        scores = jnp.sum(e * a_ref[...][None, None], axis=-1)
        m1 = jnp.maximum(
            jnp.dot(e.reshape(GB * S * S, H), w1_ref[...],
                    preferred_element_type=jnp.float32)
            + b1_ref[...], 0.0)
        msg = (jnp.dot(m1, w2_ref[...], preferred_element_type=jnp.float32)
               + b2_ref[...]).reshape(GB, S, S, H)

        mk3 = mk.reshape(GB, 1, S)

        def update(gated, m_sc, l_sc, s_sc):
            if gated:
                s = jnp.where(mk3 > 0.0, scores, NEG)
            else:
                s = scores
            m_new = jnp.maximum(m_sc[...],
                                jnp.max(s, axis=-1, keepdims=True))
            alpha = jnp.exp(m_sc[...] - m_new)
            pw = jnp.exp(s - m_new)
            if gated:
                pw = jnp.where(mk3 > 0.0, pw, 0.0)
            l_sc[...] = alpha * l_sc[...] + jnp.sum(pw, axis=-1, keepdims=True)
            s_sc[...] = alpha * s_sc[...] + jnp.sum(pw[..., None] * msg, axis=2)
            m_sc[...] = m_new

        @pl.when(diag)
        def _temporal():
            update(True, mt_sc, lt_sc, st_sc)

        @pl.when(adj_v > 0)
        def _spatial():
            update(mask_spatial, ms_sc, ls_sc, ss_sc)

    @pl.when(nk == N - 1)
    def _done():
        hq, _ = build_h(nq, vmq)
        pq = jnp.dot(hq, wqs_ref[...],
                     preferred_element_type=jnp.float32) + bqs_ref[...]
        skipv = pq[:, H:]
        at = (st_sc[...].reshape(GB * S, H)
              / jnp.maximum(lt_sc[...].reshape(GB * S, 1), 1e-9))
        as_ = (ss_sc[...].reshape(GB * S, H)
               / jnp.maximum(ls_sc[...].reshape(GB * S, 1), 1e-9))
        out = at + as_ + skipv
        mu = jnp.mean(out, axis=-1, keepdims=True)
        var = jnp.mean((out - mu) ** 2, axis=-1, keepdims=True)
        h_new = (out - mu) * jax.lax.rsqrt(var + eps) * lng_ref[...] + lnb_ref[...]
        ho_ref[...] = h_new.reshape(GB, S, H)

        r = jnp.maximum(jnp.dot(h_new, rw1_ref[...],
                                preferred_element_type=jnp.float32)
                        + rb1_ref[...], 0.0)
        r = jnp.maximum(jnp.dot(r, rw2_ref[...],
                                preferred_element_type=jnp.float32)
                        + rb2_ref[...], 0.0)
        imp = jnp.sum(r * rw3_ref[...], axis=-1, keepdims=True) + rb3_ref[...]
        imp_ref[...] = imp.reshape(GB, S, 1)


def _spin_layer(h_nm, x_nm, m_nm, adj_i32, vemb3, memb3, lp,
                *, gb, s_steps, n_nodes, mask_spatial, add_emb):
    B, T, H = h_nm.shape
    GB, S, N = gb, s_steps, n_nodes
    Cout = lp["ro_b3"].shape[-1]

    wqs = jnp.concatenate([lp["wq"], lp["wskip"]], axis=1)
    bqs = jnp.concatenate([lp["bq"], lp["bskip"]], axis=1)

    full = lambda bi, nq, nk, *_: (bi, 0, 0)
    qo = lambda bi, nq, nk, *_: (bi, nq, 0)
    eq = lambda bi, nq, nk, *_: (nq, 0, 0)
    ek = lambda bi, nq, nk, *_: (nk, 0, 0)
    c2 = lambda bi, nq, nk, *_: (0, 0)

    body = functools.partial(_layer_kernel, hidden=H, gb=GB, s_steps=S,
                             n_nodes=N, mask_spatial=mask_spatial,
                             add_emb=add_emb)

    return pl.pallas_call(
        body,
        out_shape=(jax.ShapeDtypeStruct((B, T, H), jnp.float32),
                   jax.ShapeDtypeStruct((B, T, Cout), jnp.float32)),
        grid_spec=pltpu.PrefetchScalarGridSpec(
            num_scalar_prefetch=1,
            grid=(B // GB, N, N),
            in_specs=[
                pl.BlockSpec((GB, T, H), full),
                pl.BlockSpec((GB, T, 1), full),
                pl.BlockSpec((GB, T, 1), full),
                pl.BlockSpec((1, H), c2), pl.BlockSpec((1, H), c2),
                pl.BlockSpec((H, 2 * H), c2), pl.BlockSpec((1, 2 * H), c2),
                pl.BlockSpec((H, H), c2), pl.BlockSpec((1, H), c2),
                pl.BlockSpec((H, H), c2), pl.BlockSpec((1, H), c2),
                pl.BlockSpec((H, H), c2), pl.BlockSpec((1, H), c2),
                pl.BlockSpec((1, H), c2),
                pl.BlockSpec((1, H), c2), pl.BlockSpec((1, H), c2),
                pl.BlockSpec((H, H), c2), pl.BlockSpec((1, H), c2),
                pl.BlockSpec((H, H), c2), pl.BlockSpec((1, H), c2),
                pl.BlockSpec((1, H), c2), pl.BlockSpec((1, Cout), c2),
                pl.BlockSpec((1, 1, H), eq), pl.BlockSpec((1, 1, H), eq),
                pl.BlockSpec((1, 1, H), ek), pl.BlockSpec((1, 1, H), ek),
            ],
            out_specs=(pl.BlockSpec((GB, S, H), qo),
                       pl.BlockSpec((GB, S, Cout), qo)),
            scratch_shapes=[
                pltpu.VMEM((GB, S, 1), jnp.float32),
                pltpu.VMEM((GB, S, 1), jnp.float32),
                pltpu.VMEM((GB, S, H), jnp.float32),
                pltpu.VMEM((GB, S, 1), jnp.float32),
                pltpu.VMEM((GB, S, 1), jnp.float32),
                pltpu.VMEM((GB, S, H), jnp.float32),
            ],
        ),
        compiler_params=pltpu.CompilerParams(
            dimension_semantics=("parallel", "parallel", "arbitrary"),
            vmem_limit_bytes=_VMEM_LIMIT),
    )(adj_i32, h_nm, x_nm, m_nm,
      lp["xskip_w"], lp["xskip_b"], wqs, bqs, lp["wk"], lp["bk"],
      lp["w1"], lp["b1"], lp["w2"], lp["b2"], lp["a"],
      lp["ln_g"], lp["ln_b"],
      lp["ro_w1"], lp["ro_b1"], lp["ro_w2"], lp["ro_b2"],
      lp["ro_w3"], lp["ro_b3"], vemb3, memb3, vemb3, memb3)


def _forward(p, x, u, mask, adj, *, n_layers=4, eta=3):
    B, S, N, _ = x.shape
    H = p["h_norm_g"].shape[-1]
    T = S * N
    GB = 8 if B % 8 == 0 else 1

    h = _encode(x, mask, u, p, S, N)

    h_nm = h.reshape(B, S, N, H).transpose(0, 2, 1, 3).reshape(B, T, H)
    x_nm = x.astype(jnp.float32).transpose(0, 2, 1, 3).reshape(B, T, 1)
    m_nm = mask.astype(jnp.float32).transpose(0, 2, 1, 3).reshape(B, T, 1)
    adj_i32 = (adj > 0).astype(jnp.int32).reshape(N * N)
    vemb3 = p["valid_emb"].reshape(N, 1, H)
    memb3 = p["mask_emb"].reshape(N, 1, H)

    imputations = []
    for l in range(n_layers):
        h_nm, r = _spin_layer(h_nm, x_nm, m_nm, adj_i32, vemb3, memb3,
                              p["layers"][l], gb=GB, s_steps=S, n_nodes=N,
                              mask_spatial=(l < eta), add_emb=(l == eta))
        imputations.append(
            r.reshape(B, N, S, -1).transpose(0, 2, 1, 3))
    x_hat = imputations.pop(-1)
    return x_hat, imputations


def kernel(x, u, mask, adj,
           u_lin_w, u_lin_b, u_node_emb,
           u_mlp_w1, u_mlp_b1, u_mlp_w2, u_mlp_b2, u_mlp_w3, u_mlp_b3,
           h_enc_w1, h_enc_b1, h_enc_w2, h_enc_b2,
           h_norm_g, h_norm_b, valid_emb, mask_emb,
           l0_xskip_w, l0_xskip_b, l0_wq, l0_bq, l0_wk, l0_bk, l0_w1, l0_b1, l0_w2, l0_b2,
           l0_a, l0_wskip, l0_bskip, l0_ln_g, l0_ln_b,
           l0_ro_w1, l0_ro_b1, l0_ro_w2, l0_ro_b2, l0_ro_w3, l0_ro_b3,
           l1_xskip_w, l1_xskip_b, l1_wq, l1_bq, l1_wk, l1_bk, l1_w1, l1_b1, l1_w2, l1_b2,
           l1_a, l1_wskip, l1_bskip, l1_ln_g, l1_ln_b,
           l1_ro_w1, l1_ro_b1, l1_ro_w2, l1_ro_b2, l1_ro_w3, l1_ro_b3,
           l2_xskip_w, l2_xskip_b, l2_wq, l2_bq, l2_wk, l2_bk, l2_w1, l2_b1, l2_w2, l2_b2,
           l2_a, l2_wskip, l2_bskip, l2_ln_g, l2_ln_b,
           l2_ro_w1, l2_ro_b1, l2_ro_w2, l2_ro_b2, l2_ro_w3, l2_ro_b3,
           l3_xskip_w, l3_xskip_b, l3_wq, l3_bq, l3_wk, l3_bk, l3_w1, l3_b1, l3_w2, l3_b2,
           l3_a, l3_wskip, l3_bskip, l3_ln_g, l3_ln_b,
           l3_ro_w1, l3_ro_b1, l3_ro_w2, l3_ro_b2, l3_ro_w3, l3_ro_b3):
    def _layer(pp):
        return {"xskip_w": pp[0], "xskip_b": pp[1], "wq": pp[2], "bq": pp[3],
                "wk": pp[4], "bk": pp[5], "w1": pp[6], "b1": pp[7], "w2": pp[8],
                "b2": pp[9], "a": pp[10], "wskip": pp[11], "bskip": pp[12],
                "ln_g": pp[13], "ln_b": pp[14], "ro_w1": pp[15], "ro_b1": pp[16],
                "ro_w2": pp[17], "ro_b2": pp[18], "ro_w3": pp[19], "ro_b3": pp[20]}

    p = {
        "u_lin_w": u_lin_w, "u_lin_b": u_lin_b, "u_node_emb": u_node_emb,
        "u_mlp_w1": u_mlp_w1, "u_mlp_b1": u_mlp_b1, "u_mlp_w2": u_mlp_w2,
        "u_mlp_b2": u_mlp_b2, "u_mlp_w3": u_mlp_w3, "u_mlp_b3": u_mlp_b3,
        "h_enc_w1": h_enc_w1, "h_enc_b1": h_enc_b1, "h_enc_w2": h_enc_w2,
        "h_enc_b2": h_enc_b2, "h_norm_g": h_norm_g, "h_norm_b": h_norm_b,
        "valid_emb": valid_emb, "mask_emb": mask_emb,
        "layers": [
            _layer([l0_xskip_w, l0_xskip_b, l0_wq, l0_bq, l0_wk, l0_bk, l0_w1,
                    l0_b1, l0_w2, l0_b2, l0_a, l0_wskip, l0_bskip, l0_ln_g,
                    l0_ln_b, l0_ro_w1, l0_ro_b1, l0_ro_w2, l0_ro_b2, l0_ro_w3,
                    l0_ro_b3]),
            _layer([l1_xskip_w, l1_xskip_b, l1_wq, l1_bq, l1_wk, l1_bk, l1_w1,
                    l1_b1, l1_w2, l1_b2, l1_a, l1_wskip, l1_bskip, l1_ln_g,
                    l1_ln_b, l1_ro_w1, l1_ro_b1, l1_ro_w2, l1_ro_b2, l1_ro_w3,
                    l1_ro_b3]),
            _layer([l2_xskip_w, l2_xskip_b, l2_wq, l2_bq, l2_wk, l2_bk, l2_w1,
                    l2_b1, l2_w2, l2_b2, l2_a, l2_wskip, l2_bskip, l2_ln_g,
                    l2_ln_b, l2_ro_w1, l2_ro_b1, l2_ro_w2, l2_ro_b2, l2_ro_w3,
                    l2_ro_b3]),
            _layer([l3_xskip_w, l3_xskip_b, l3_wq, l3_bq, l3_wk, l3_bk, l3_w1,
                    l3_b1, l3_w2, l3_b2, l3_a, l3_wskip, l3_bskip, l3_ln_g,
                    l3_ln_b, l3_ro_w1, l3_ro_b1, l3_ro_w2, l3_ro_b2, l3_ro_w3,
                    l3_ro_b3]),
        ],
    }
    return _forward(p, x, u, mask, adj, n_layers=4, eta=3)
